# Initial kernel scaffold; baseline (speedup 1.0000x reference)
import jax
import jax.numpy as jnp
from jax.experimental import pallas as pl


def kernel(A, B, index):
    raise NotImplementedError("write your pallas kernel here")


# speedup vs baseline: 29.4005x; 29.4005x over previous
import functools

import jax
import jax.numpy as jnp
from jax import lax
from jax.experimental import pallas as pl
from jax.experimental.pallas import tpu as pltpu
from jax.experimental.pallas import tpu_sc as plsc

_B, _N, _K, _D = 2, 2048, 32, 1024
_ROWS = _B * _N
_NW = 32
_RPW = _ROWS // _NW
_RBLK = 32


def _mm_body(a_ref, b_ref, s_ref):
    s_ref[...] = lax.dot_general(
        a_ref[0], b_ref[0],
        dimension_numbers=(((1,), (1,)), ((), ())),
        preferred_element_type=jnp.float32,
    )[None]


def _dense_scores(A, B):
    bm = 256
    return pl.pallas_call(
        _mm_body,
        grid=(_B, _N // bm),
        in_specs=[
            pl.BlockSpec((1, bm, _D), lambda b, m: (b, m, 0)),
            pl.BlockSpec((1, _N, _D), lambda b, m: (b, 0, 0)),
        ],
        out_specs=pl.BlockSpec((1, bm, _N), lambda b, m: (b, m, 0)),
        out_shape=jax.ShapeDtypeStruct((_B, _N, _N), jnp.float32),
    )(A, B)


def _sc_sample_body(s_hbm, idx_hbm, out_hbm, s_v, idx_v, out_v):
    wid = lax.axis_index("s") * 2 + lax.axis_index("c")
    row0 = wid * _RPW

    def do_block(blk, _):
        base = row0 + blk * _RBLK
        pltpu.sync_copy(s_hbm.at[pl.ds(base, _RBLK)], s_v)
        pltpu.sync_copy(idx_hbm.at[pl.ds(base, _RBLK)], idx_v)

        def do_row(r, _):
            rvec = jnp.broadcast_to(r, (16,)).astype(jnp.int32)
            for h in range(_K // 16):
                iv = idx_v[r, pl.ds(h * 16, 16)]
                out_v[r, pl.ds(h * 16, 16)] = plsc.load_gather(s_v, [rvec, iv])
            return 0

        lax.fori_loop(0, _RBLK, do_row, 0)
        pltpu.sync_copy(out_v, out_hbm.at[pl.ds(base, _RBLK)])
        return 0

    lax.fori_loop(0, _RPW // _RBLK, do_block, 0)


def _sc_sample(S2, idx2):
    mesh = plsc.VectorSubcoreMesh(core_axis_name="c", subcore_axis_name="s")
    return pl.kernel(
        _sc_sample_body,
        out_type=jax.ShapeDtypeStruct((_ROWS, _K), jnp.float32),
        mesh=mesh,
        scratch_types=[
            pltpu.VMEM((_RBLK, _N), jnp.float32),
            pltpu.VMEM((_RBLK, _K), jnp.int32),
            pltpu.VMEM((_RBLK, _K), jnp.float32),
        ],
        compiler_params=pltpu.CompilerParams(
            use_tc_tiling_on_sc=False, needs_layout_passes=False),
    )(S2, idx2)


def kernel(A, B, index):
    S = _dense_scores(A, B)
    S2 = S.reshape(_ROWS, _N)
    idx2 = index.reshape(_ROWS, _K)
    C2 = _sc_sample(S2, idx2)
    return C2.reshape(_B, _N, _K)


# speedup vs baseline: 38.1873x; 1.2989x over previous
import functools

import jax
import jax.numpy as jnp
from jax import lax
from jax.experimental import pallas as pl
from jax.experimental.pallas import tpu as pltpu
from jax.experimental.pallas import tpu_sc as plsc

_B, _N, _K, _D = 2, 2048, 32, 1024
_ROWS = _B * _N
_NW = 32
_RPW = _ROWS // _NW
_RBLK = 32


def _mm_body(a_ref, b_ref, s_ref):
    s_ref[...] = lax.dot_general(
        a_ref[0], b_ref[0],
        dimension_numbers=(((1,), (1,)), ((), ())),
        preferred_element_type=jnp.float32,
    )[None]


def _dense_scores(A, B):
    bm = 256
    return pl.pallas_call(
        _mm_body,
        grid=(_B, _N // bm),
        in_specs=[
            pl.BlockSpec((1, bm, _D), lambda b, m: (b, m, 0)),
            pl.BlockSpec((1, _N, _D), lambda b, m: (b, 0, 0)),
        ],
        out_specs=pl.BlockSpec((1, bm, _N), lambda b, m: (b, m, 0)),
        out_shape=jax.ShapeDtypeStruct((_B, _N, _N), jnp.float32),
    )(A, B)


def _sc_sample_body(s_hbm, idx_hbm, out_hbm, s_v, idx_v, out_v):
    wid = lax.axis_index("s") * 2 + lax.axis_index("c")
    row0 = wid * _RPW

    def do_block(blk, _):
        base = row0 + blk * _RBLK
        pltpu.sync_copy(s_hbm.at[pl.ds(base, _RBLK)], s_v)
        pltpu.sync_copy(idx_hbm.at[pl.ds(base, _RBLK)], idx_v)

        def do_row(r, _):
            rvec = jnp.broadcast_to(r, (16,)).astype(jnp.int32)
            for h in range(_K // 16):
                iv = idx_v[r, pl.ds(h * 16, 16)]
                out_v[r, pl.ds(h * 16, 16)] = plsc.load_gather(s_v, [rvec, iv])
            return 0

        lax.fori_loop(0, _RBLK, do_row, 0)
        pltpu.sync_copy(out_v, out_hbm.at[pl.ds(base, _RBLK)])
        return 0

    lax.fori_loop(0, _RPW // _RBLK, do_block, 0)


def _sc_sample(S2, idx2):
    mesh = plsc.VectorSubcoreMesh(core_axis_name="c", subcore_axis_name="s")
    return pl.kernel(
        _sc_sample_body,
        out_type=jax.ShapeDtypeStruct((_ROWS, _K), jnp.float32),
        mesh=mesh,
        scratch_types=[
            pltpu.VMEM((_RBLK, _N), jnp.float32),
            pltpu.VMEM((_RBLK, _K), jnp.int32),
            pltpu.VMEM((_RBLK, _K), jnp.float32),
        ],
        compiler_params=pltpu.CompilerParams(
            use_tc_tiling_on_sc=True, needs_layout_passes=False),
    )(S2, idx2)


def kernel(A, B, index):
    S = _dense_scores(A, B)
    S2 = S.reshape(_ROWS, _N)
    idx2 = index.reshape(_ROWS, _K)
    C2 = _sc_sample(S2, idx2)
    return C2.reshape(_B, _N, _K)


# speedup vs baseline: 38.1909x; 1.0001x over previous
import functools

import jax
import jax.numpy as jnp
from jax import lax
from jax.experimental import pallas as pl
from jax.experimental.pallas import tpu as pltpu
from jax.experimental.pallas import tpu_sc as plsc

_B, _N, _K, _D = 2, 2048, 32, 1024
_ROWS = _B * _N
_NW = 32
_RPW = _ROWS // _NW
_RBLK = 32


def _mm_body(a_ref, b_ref, s_ref):
    s_ref[...] = lax.dot_general(
        a_ref[0].astype(jnp.bfloat16), b_ref[0].astype(jnp.bfloat16),
        dimension_numbers=(((1,), (1,)), ((), ())),
        preferred_element_type=jnp.float32,
    )[None]


def _dense_scores(A, B):
    bm = 256
    return pl.pallas_call(
        _mm_body,
        grid=(_B, _N // bm),
        in_specs=[
            pl.BlockSpec((1, bm, _D), lambda b, m: (b, m, 0)),
            pl.BlockSpec((1, _N, _D), lambda b, m: (b, 0, 0)),
        ],
        out_specs=pl.BlockSpec((1, bm, _N), lambda b, m: (b, m, 0)),
        out_shape=jax.ShapeDtypeStruct((_B, _N, _N), jnp.float32),
    )(A, B)


def _sc_sample_body(s_hbm, idx_hbm, out_hbm, s_v, idx_v, out_v):
    wid = lax.axis_index("s") * 2 + lax.axis_index("c")
    row0 = wid * _RPW

    def do_block(blk, _):
        base = row0 + blk * _RBLK
        pltpu.sync_copy(s_hbm.at[pl.ds(base, _RBLK)], s_v)
        pltpu.sync_copy(idx_hbm.at[pl.ds(base, _RBLK)], idx_v)

        def do_row(r, _):
            rvec = jnp.broadcast_to(r, (16,)).astype(jnp.int32)
            for h in range(_K // 16):
                iv = idx_v[r, pl.ds(h * 16, 16)]
                out_v[r, pl.ds(h * 16, 16)] = plsc.load_gather(s_v, [rvec, iv])
            return 0

        lax.fori_loop(0, _RBLK, do_row, 0)
        pltpu.sync_copy(out_v, out_hbm.at[pl.ds(base, _RBLK)])
        return 0

    lax.fori_loop(0, _RPW // _RBLK, do_block, 0)


def _sc_sample(S2, idx2):
    mesh = plsc.VectorSubcoreMesh(core_axis_name="c", subcore_axis_name="s")
    return pl.kernel(
        _sc_sample_body,
        out_type=jax.ShapeDtypeStruct((_ROWS, _K), jnp.float32),
        mesh=mesh,
        scratch_types=[
            pltpu.VMEM((_RBLK, _N), jnp.float32),
            pltpu.VMEM((_RBLK, _K), jnp.int32),
            pltpu.VMEM((_RBLK, _K), jnp.float32),
        ],
        compiler_params=pltpu.CompilerParams(
            use_tc_tiling_on_sc=True, needs_layout_passes=False),
    )(S2, idx2)


def kernel(A, B, index):
    S = _dense_scores(A, B)
    S2 = S.reshape(_ROWS, _N)
    idx2 = index.reshape(_ROWS, _K)
    C2 = _sc_sample(S2, idx2)
    return C2.reshape(_B, _N, _K)
